# baseline (speedup 1.0000x reference)
import functools

import jax
import jax.numpy as jnp
from jax import lax
from jax.experimental import pallas as pl
from jax.experimental.pallas import tpu as pltpu
from jax.experimental.pallas import tpu_sc as plsc

NW = 32
L = 16
CH = 80
NBUF = 8
TBL_ROWS = 1792


def kernel(node_type, edge_index, table):
    n_nodes = node_type.shape[0]
    n_edges = edge_index.shape[1]
    n_rows, d = table.shape

    per_w = n_edges // NW
    assert per_w * NW == n_edges and per_w % L == 0
    n_ch = per_w // CH
    assert n_ch * CH == per_w
    pre = n_ch % NBUF
    n_trips = n_ch // NBUF
    grp_per_ch = CH // L
    assert grp_per_ch * L == CH
    offs = [(w * per_w) % 128 for w in range(NW)]
    win = -(-(per_w + max(offs)) // 128) * 128
    assert all(o % L == 0 for o in offs)
    assert (NW - 1) * per_w - offs[-1] + win <= n_edges

    mesh = plsc.VectorSubcoreMesh(core_axis_name="c", subcore_axis_name="s")

    @functools.partial(
        pl.kernel,
        mesh=mesh,
        out_type=jax.ShapeDtypeStruct((n_edges, d), jnp.float32),
        compiler_params=pltpu.CompilerParams(needs_layout_passes=False),
        scratch_types=[
            pltpu.VMEM((n_nodes,), jnp.int32),
            pltpu.VMEM((2, win), jnp.int32),
            pltpu.VMEM((NBUF, CH), jnp.int32),
            pltpu.VMEM((NBUF, CH, d), jnp.float32),
            pltpu.VMEM_SHARED((TBL_ROWS, d), jnp.float32),
            [pltpu.SemaphoreType.DMA] * NBUF,
            [pltpu.SemaphoreType.DMA] * NBUF,
            pltpu.SemaphoreType.DMA,
            pltpu.SemaphoreType.DMA,
        ],
    )
    def sc_kernel(nt_hbm, ei_hbm, tbl_hbm, out_hbm, nt_v, ed_v, et_v,
                  rows_v, tbl_s, gsem, wsem, ssem, tsem):
        sid = lax.axis_index("s")
        wid = sid * 2 + lax.axis_index("c")
        base = wid * per_w
        start = (base // 128) * 128
        off = base - start

        stripe = TBL_ROWS // 16
        pltpu.async_copy(tbl_hbm.at[pl.ds(sid * stripe, stripe)],
                         tbl_s.at[pl.ds(sid * stripe, stripe)], tsem)
        pltpu.async_copy(nt_hbm, nt_v, ssem)
        pltpu.async_copy(ei_hbm.at[:, pl.ds(start, win)], ed_v, ssem)
        pltpu.make_async_copy(nt_hbm, nt_v, ssem).wait()
        pltpu.make_async_copy(ei_hbm.at[:, pl.ds(start, win)], ed_v, ssem).wait()

        def compute_row(j, slot):
            for g in range(grp_per_ch):
                p = off + (j * grp_per_ch + g) * L
                ts = plsc.load_gather(nt_v, [ed_v[0, pl.ds(p, L)]])
                td = plsc.load_gather(nt_v, [ed_v[1, pl.ds(p, L)]])
                s = ts + td
                et_v[slot, pl.ds(g * L, L)] = ((s * (s + 1)) >> 1) + td

        def start_gather(b):
            pltpu.async_copy(tbl_s.at[et_v.at[b]], rows_v.at[b], gsem[b])

        def wait_gather(b):
            pltpu.make_async_copy(tbl_s.at[et_v.at[0]], rows_v.at[b],
                                  gsem[b]).wait()

        def start_write(j, b):
            pltpu.async_copy(rows_v.at[b],
                             out_hbm.at[pl.ds(base + j * CH, CH)], wsem[b])

        def wait_write(b):
            pltpu.make_async_copy(rows_v.at[b],
                                  out_hbm.at[pl.ds(base, CH)], wsem[b]).wait()

        for j in range(NBUF):
            compute_row(j, j)
        pltpu.make_async_copy(tbl_hbm.at[pl.ds(0, stripe)],
                              tbl_s.at[pl.ds(0, stripe)], tsem).wait()
        plsc.subcore_barrier()
        for j in range(pre):
            start_gather(j)
        for j in range(pre):
            wait_gather(j)
            start_write(j, j)
        for j in range(pre, NBUF):
            start_gather(j)
        for j in range(NBUF, pre + NBUF):
            s = j % NBUF
            compute_row(j, s)
            wait_write(s)
            start_gather(s)
        for j in range(pre, pre + NBUF):
            s = j % NBUF
            wait_gather(s)
            start_write(j, s)

        def trip(t, carry):
            a = pre + t * NBUF
            for b in range(NBUF):
                j = a + b
                s = (pre + b) % NBUF
                compute_row(j, s)
                wait_write(s)
                start_gather(s)
            for b in range(NBUF):
                j = a + b
                s = (pre + b) % NBUF
                wait_gather(s)
                start_write(j, s)
            return carry

        lax.fori_loop(1, n_trips, trip, 0)

        for b in range(NBUF):
            wait_write(b)

    return sc_kernel(node_type, edge_index, table)


# speedup vs baseline: 1.0131x; 1.0131x over previous
import functools

import jax
import jax.numpy as jnp
from jax import lax
from jax.experimental import pallas as pl
from jax.experimental.pallas import tpu as pltpu
from jax.experimental.pallas import tpu_sc as plsc

NW = 32
L = 16
CH = 80
NBUF = 5
TBL_ROWS = 1792


def kernel(node_type, edge_index, table):
    n_nodes = node_type.shape[0]
    n_edges = edge_index.shape[1]
    n_rows, d = table.shape

    per_w = n_edges // NW
    assert per_w * NW == n_edges and per_w % L == 0
    n_ch = per_w // CH
    assert n_ch * CH == per_w and n_ch % NBUF == 0
    grp_per_ch = CH // L
    assert grp_per_ch * L == CH
    offs = [(w * per_w) % 128 for w in range(NW)]
    win = -(-(per_w + max(offs)) // 128) * 128
    assert all(o % L == 0 for o in offs)
    assert (NW - 1) * per_w - offs[-1] + win <= n_edges

    mesh = plsc.VectorSubcoreMesh(core_axis_name="c", subcore_axis_name="s")

    @functools.partial(
        pl.kernel,
        mesh=mesh,
        out_type=jax.ShapeDtypeStruct((n_edges, d), jnp.float32),
        compiler_params=pltpu.CompilerParams(needs_layout_passes=False),
        scratch_types=[
            pltpu.VMEM((n_nodes,), jnp.int32),
            pltpu.VMEM((2, win), jnp.int32),
            pltpu.VMEM((n_ch, CH), jnp.int32),
            pltpu.VMEM((NBUF, CH, d), jnp.float32),
            pltpu.VMEM_SHARED((TBL_ROWS, d), jnp.float32),
            [pltpu.SemaphoreType.DMA] * NBUF,
            [pltpu.SemaphoreType.DMA] * NBUF,
            pltpu.SemaphoreType.DMA,
            pltpu.SemaphoreType.DMA,
        ],
    )
    def sc_kernel(nt_hbm, ei_hbm, tbl_hbm, out_hbm, nt_v, ed_v, et_v,
                  rows_v, tbl_s, gsem, wsem, ssem, tsem):
        sid = lax.axis_index("s")
        wid = sid * 2 + lax.axis_index("c")
        base = wid * per_w
        start = (base // 128) * 128
        off = base - start

        stripe = TBL_ROWS // 16
        pltpu.async_copy(tbl_hbm.at[pl.ds(sid * stripe, stripe)],
                         tbl_s.at[pl.ds(sid * stripe, stripe)], tsem)
        pltpu.async_copy(nt_hbm, nt_v, ssem)
        pltpu.async_copy(ei_hbm.at[:, pl.ds(start, win)], ed_v, ssem)
        pltpu.make_async_copy(nt_hbm, nt_v, ssem).wait()
        pltpu.make_async_copy(ei_hbm.at[:, pl.ds(start, win)], ed_v, ssem).wait()

        def compute_row(j):
            for g in range(grp_per_ch):
                p = off + (j * grp_per_ch + g) * L
                ts = plsc.load_gather(nt_v, [ed_v[0, pl.ds(p, L)]])
                td = plsc.load_gather(nt_v, [ed_v[1, pl.ds(p, L)]])
                s = ts + td
                et_v[j, pl.ds(g * L, L)] = ((s * (s + 1)) >> 1) + td

        def start_gather(j, b):
            pltpu.async_copy(tbl_s.at[et_v.at[j]], rows_v.at[b], gsem[b])

        def wait_gather(b):
            pltpu.make_async_copy(tbl_s.at[et_v.at[0]], rows_v.at[b],
                                  gsem[b]).wait()

        def start_write(j, b):
            pltpu.async_copy(rows_v.at[b],
                             out_hbm.at[pl.ds(base + j * CH, CH)], wsem[b])

        def wait_write(b):
            pltpu.make_async_copy(rows_v.at[b],
                                  out_hbm.at[pl.ds(base, CH)], wsem[b]).wait()

        for b in range(NBUF):
            compute_row(b)
        pltpu.make_async_copy(tbl_hbm.at[pl.ds(0, stripe)],
                              tbl_s.at[pl.ds(0, stripe)], tsem).wait()
        plsc.subcore_barrier()
        for b in range(NBUF):
            start_gather(b, b)
        for b in range(NBUF):
            wait_gather(b)
            start_write(b, b)

        def trip(t, carry):
            a = t * NBUF
            for b in range(NBUF):
                compute_row(a + b)
                wait_write(b)
                start_gather(a + b, b)
            for b in range(NBUF):
                wait_gather(b)
                start_write(a + b, b)
            return carry

        lax.fori_loop(1, n_ch // NBUF, trip, 0)

        for b in range(NBUF):
            wait_write(b)

    return sc_kernel(node_type, edge_index, table)
